# Initial kernel scaffold; baseline (speedup 1.0000x reference)
import functools

import jax
import jax.numpy as jnp
from jax import lax
from jax.experimental import pallas as pl
from jax.experimental.pallas import tpu as pltpu
from jax.experimental.pallas import tpu_sc as plsc

_BATCH = 16384
_N_WORDS = 50
_TOKEN_SPACE = 100000
_N_OOV = 100
_V = 1001
_VPAD = 1008
_L = 16
_NC = 2
_NS = 16
_NW = _NC * _NS
_TOTAL = _BATCH * _N_WORDS
_PER_W = _TOTAL // _NW
_CHUNK = 12800
_NCHUNK = _PER_W // _CHUNK

_mesh = plsc.VectorSubcoreMesh(core_axis_name="c", subcore_axis_name="s")


@functools.partial(
    pl.kernel,
    mesh=_mesh,
    out_type=jax.ShapeDtypeStruct((_TOTAL,), jnp.int32),
    scratch_types=[
        pltpu.VMEM((_TOKEN_SPACE,), jnp.int32),
        pltpu.VMEM((_VPAD,), jnp.int32),
        pltpu.VMEM((_CHUNK,), jnp.int32),
        pltpu.VMEM((_CHUNK,), jnp.int32),
    ],
)
def _lookup(tok_hbm, keys_hbm, out_hbm, table, keys_v, tok_v, res_v):
    wid = lax.axis_index("s") * _NC + lax.axis_index("c")
    base = wid * _PER_W

    pltpu.sync_copy(keys_hbm, keys_v)

    def scatter_body(j, carry):
        lanes = lax.iota(jnp.int32, _L) + j * _L
        ranks = jnp.minimum(lanes, _V - 1)
        keys = keys_v[pl.ds(j * _L, _L)]
        plsc.store_scatter(table, [keys], ranks)
        return carry

    lax.fori_loop(0, _VPAD // _L, scatter_body, 0)

    for c in range(_NCHUNK):
        start = base + c * _CHUNK
        pltpu.sync_copy(tok_hbm.at[pl.ds(start, _CHUNK)], tok_v)

        def vec_body(i, carry):
            t = tok_v[pl.ds(i * _L, _L)]
            g = plsc.load_gather(table, [t])
            gc = jnp.clip(g, 0, _V - 1)
            k = plsc.load_gather(keys_v, [gc])
            oov = _V + lax.rem(t, _N_OOV)
            res_v[pl.ds(i * _L, _L)] = jnp.where(k == t, gc, oov)
            return carry

        lax.fori_loop(0, _CHUNK // _L, vec_body, 0)

        pltpu.sync_copy(res_v, out_hbm.at[pl.ds(start, _CHUNK)])


def kernel(inputs, vocab_keys):
    tok = inputs.reshape(-1).astype(jnp.int32)
    keys = vocab_keys.astype(jnp.int32)
    keys = jnp.concatenate(
        [keys, jnp.full((_VPAD - _V,), keys[_V - 1], jnp.int32)]
    )
    out = _lookup(tok, keys)
    return out.reshape(_BATCH, _N_WORDS).astype(inputs.dtype)


# speedup vs baseline: 663.7834x; 663.7834x over previous
import functools

import jax
import jax.numpy as jnp
from jax import lax
from jax.experimental import pallas as pl
from jax.experimental.pallas import tpu as pltpu
from jax.experimental.pallas import tpu_sc as plsc

_BATCH = 16384
_N_WORDS = 50
_TOKEN_SPACE = 100000
_N_OOV = 100
_V = 1001
_VPAD = 1008
_L = 16
_NC = 2
_NS = 16
_NW = _NC * _NS
_TOTAL = _BATCH * _N_WORDS
_PER_W = _TOTAL // _NW
_CHUNK = 12800
_NCHUNK = _PER_W // _CHUNK

_mesh = plsc.VectorSubcoreMesh(core_axis_name="c", subcore_axis_name="s")


@functools.partial(
    pl.kernel,
    mesh=_mesh,
    out_type=jax.ShapeDtypeStruct((_TOTAL,), jnp.int32),
    scratch_types=[
        pltpu.VMEM((_TOKEN_SPACE,), jnp.int32),
        pltpu.VMEM((_VPAD,), jnp.int32),
        pltpu.VMEM((_CHUNK,), jnp.int32),
        pltpu.VMEM((_CHUNK,), jnp.int32),
    ],
    compiler_params=pltpu.CompilerParams(needs_layout_passes=False),
)
def _lookup(tok_hbm, keys_hbm, out_hbm, table, keys_v, tok_v, res_v):
    wid = lax.axis_index("s") * _NC + lax.axis_index("c")
    base = wid * _PER_W

    pltpu.sync_copy(keys_hbm, keys_v)

    def scatter_body(j, off):
        lanes = lax.iota(jnp.int32, _L) + off
        ranks = jnp.minimum(lanes, jnp.int32(_V - 1))
        keys = keys_v[pl.ds(off, _L)]
        plsc.store_scatter(table, [keys], ranks)
        return off + jnp.int32(_L)

    lax.fori_loop(0, _VPAD // _L, scatter_body, jnp.int32(0))

    for c in range(_NCHUNK):
        start = base + c * _CHUNK
        pltpu.sync_copy(tok_hbm.at[pl.ds(start, _CHUNK)], tok_v)

        def vec_body(i, off):
            t = tok_v[pl.ds(off, _L)]
            g = plsc.load_gather(table, [t])
            gc = jnp.clip(g, jnp.int32(0), jnp.int32(_V - 1))
            k = plsc.load_gather(keys_v, [gc])
            oov = jnp.int32(_V) + lax.rem(t, jnp.int32(_N_OOV))
            res_v[pl.ds(off, _L)] = jnp.where(k == t, gc, oov)
            return off + jnp.int32(_L)

        lax.fori_loop(0, _CHUNK // _L, vec_body, jnp.int32(0))

        pltpu.sync_copy(res_v, out_hbm.at[pl.ds(start, _CHUNK)])


def kernel(inputs, vocab_keys):
    tok = inputs.reshape(-1).astype(jnp.int32)
    keys = vocab_keys.astype(jnp.int32)
    keys = jnp.concatenate(
        [keys, jnp.full((_VPAD - _V,), keys[_V - 1], jnp.int32)]
    )
    out = _lookup(tok, keys)
    return out.reshape(_BATCH, _N_WORDS).astype(inputs.dtype)


# speedup vs baseline: 1338.3269x; 2.0162x over previous
import functools

import jax
import jax.numpy as jnp
from jax import lax
from jax.experimental import pallas as pl
from jax.experimental.pallas import tpu as pltpu
from jax.experimental.pallas import tpu_sc as plsc

_BATCH = 16384
_N_WORDS = 50
_TOKEN_SPACE = 100000
_N_OOV = 100
_V = 1001
_VPAD = 1008
_L = 16
_NC = 2
_NS = 16
_NW = _NC * _NS
_COLS_PER_W = _BATCH // _NW
_CCOLS = 256
_NCHUNK = _COLS_PER_W // _CCOLS

_mesh = plsc.VectorSubcoreMesh(core_axis_name="c", subcore_axis_name="s")


@functools.partial(
    pl.kernel,
    mesh=_mesh,
    out_type=jax.ShapeDtypeStruct((_N_WORDS, _BATCH), jnp.int32),
    scratch_types=[
        pltpu.VMEM((_TOKEN_SPACE,), jnp.int32),
        pltpu.VMEM((_VPAD,), jnp.int32),
        pltpu.VMEM((_N_WORDS, _CCOLS), jnp.int32),
        pltpu.VMEM((_N_WORDS, _CCOLS), jnp.int32),
    ],
    compiler_params=pltpu.CompilerParams(needs_layout_passes=False),
)
def _lookup(tok_hbm, keys_hbm, out_hbm, table, keys_v, tok_v, res_v):
    wid = lax.axis_index("s") * _NC + lax.axis_index("c")

    pltpu.sync_copy(keys_hbm, keys_v)

    def scatter_body(j, off):
        lanes = lax.iota(jnp.int32, _L) + off
        ranks = jnp.minimum(lanes, jnp.int32(_V - 1))
        keys = keys_v[pl.ds(off, _L)]
        plsc.store_scatter(table, [keys], ranks)
        return off + jnp.int32(_L)

    lax.fori_loop(0, _VPAD // _L, scatter_body, jnp.int32(0))

    for c in range(_NCHUNK):
        bstart = wid * _COLS_PER_W + c * _CCOLS
        pltpu.sync_copy(tok_hbm.at[:, pl.ds(bstart, _CCOLS)], tok_v)

        def w_body(w, woff):
            for kb in range(_CCOLS // _L):
                t = tok_v[woff, pl.ds(kb * _L, _L)]
                g = plsc.load_gather(table, [t])
                gc = jnp.clip(g, jnp.int32(0), jnp.int32(_V - 1))
                k = plsc.load_gather(keys_v, [gc])
                oov = jnp.int32(_V) + lax.rem(t, jnp.int32(_N_OOV))
                res_v[woff, pl.ds(kb * _L, _L)] = jnp.where(k == t, gc, oov)
            return woff + jnp.int32(1)

        lax.fori_loop(0, _N_WORDS, w_body, jnp.int32(0))

        pltpu.sync_copy(res_v, out_hbm.at[:, pl.ds(bstart, _CCOLS)])


def kernel(inputs, vocab_keys):
    tok = inputs.astype(jnp.int32).T
    keys = vocab_keys.astype(jnp.int32)
    keys = jnp.concatenate(
        [keys, jnp.full((_VPAD - _V,), keys[_V - 1], jnp.int32)]
    )
    return _lookup(tok, keys).T.astype(jnp.int64)


# speedup vs baseline: 2202.8285x; 1.6460x over previous
import functools

import jax
import jax.numpy as jnp
from jax import lax
from jax.experimental import pallas as pl
from jax.experimental.pallas import tpu as pltpu
from jax.experimental.pallas import tpu_sc as plsc

_BATCH = 16384
_N_WORDS = 50
_TOKEN_SPACE = 100000
_N_OOV = 100
_V = 1001
_VPAD = 1008
_L = 16
_NC = 2
_NS = 16
_NW = _NC * _NS
_COLS_PER_W = _BATCH // _NW
_CCOLS = 256
_NCHUNK = _COLS_PER_W // _CCOLS

_mesh = plsc.VectorSubcoreMesh(core_axis_name="c", subcore_axis_name="s")


@functools.partial(
    pl.kernel,
    mesh=_mesh,
    out_type=jax.ShapeDtypeStruct((_N_WORDS, _BATCH), jnp.int32),
    scratch_types=[
        pltpu.VMEM((_TOKEN_SPACE,), jnp.int32),
        pltpu.VMEM((_VPAD,), jnp.int32),
        pltpu.VMEM((_N_WORDS, _CCOLS), jnp.int32),
        pltpu.VMEM((_N_WORDS, _CCOLS), jnp.int32),
    ],
    compiler_params=pltpu.CompilerParams(needs_layout_passes=False),
)
def _lookup(tok_hbm, keys_hbm, out_hbm, table, keys_v, tok_v, res_v):
    wid = lax.axis_index("s") * _NC + lax.axis_index("c")

    pltpu.sync_copy(keys_hbm, keys_v)

    def scatter_body(j, off):
        lanes = lax.iota(jnp.int32, _L) + off
        ranks = jnp.minimum(lanes, jnp.int32(_V - 1))
        keys = keys_v[pl.ds(off, _L)]
        plsc.store_scatter(table, [keys], ranks)
        return off + jnp.int32(_L)

    lax.fori_loop(0, _VPAD // _L, scatter_body, jnp.int32(0))

    for c in range(_NCHUNK):
        bstart = wid * _COLS_PER_W + c * _CCOLS
        pltpu.sync_copy(tok_hbm.at[:, pl.ds(bstart, _CCOLS)], tok_v)

        def w_body(w, woff):
            for kb in range(_CCOLS // _L):
                t = tok_v[woff, pl.ds(kb * _L, _L)]
                g = plsc.load_gather(table, [t])
                gc = jnp.clip(g, jnp.int32(0), jnp.int32(_V - 1))
                k = plsc.load_gather(keys_v, [gc])
                tf = t.astype(jnp.float32)
                q = (tf * jnp.float32(1.0 / _N_OOV)).astype(jnp.int32)
                r = (
                    tf - q.astype(jnp.float32) * jnp.float32(_N_OOV)
                ).astype(jnp.int32)
                r = jnp.where(r < 0, r + jnp.int32(_N_OOV), r)
                r = jnp.where(
                    r >= jnp.int32(_N_OOV), r - jnp.int32(_N_OOV), r
                )
                oov = jnp.int32(_V) + r
                res_v[woff, pl.ds(kb * _L, _L)] = jnp.where(k == t, gc, oov)
            return woff + jnp.int32(1)

        lax.fori_loop(0, _N_WORDS, w_body, jnp.int32(0))

        pltpu.sync_copy(res_v, out_hbm.at[:, pl.ds(bstart, _CCOLS)])


def kernel(inputs, vocab_keys):
    tok = inputs.astype(jnp.int32).T
    keys = vocab_keys.astype(jnp.int32)
    keys = jnp.concatenate(
        [keys, jnp.full((_VPAD - _V,), keys[_V - 1], jnp.int32)]
    )
    return _lookup(tok, keys).T.astype(jnp.int64)


# speedup vs baseline: 2427.7966x; 1.1021x over previous
import functools

import jax
import jax.numpy as jnp
from jax import lax
from jax.experimental import pallas as pl
from jax.experimental.pallas import tpu as pltpu
from jax.experimental.pallas import tpu_sc as plsc

_BATCH = 16384
_N_WORDS = 50
_TOKEN_SPACE = 100000
_N_OOV = 100
_V = 1001
_VPAD = 1008
_L = 16
_NC = 2
_NS = 16
_NW = _NC * _NS
_COLS_PER_W = _BATCH // _NW
_CCOLS = 128
_NCHUNK = _COLS_PER_W // _CCOLS
_INIT_UNROLL = 10

_mesh = plsc.VectorSubcoreMesh(core_axis_name="c", subcore_axis_name="s")


@functools.partial(
    pl.kernel,
    mesh=_mesh,
    out_type=jax.ShapeDtypeStruct((_N_WORDS, _BATCH), jnp.int32),
    scratch_types=[
        pltpu.VMEM((_TOKEN_SPACE,), jnp.int32),
        pltpu.VMEM((_VPAD,), jnp.int32),
        pltpu.VMEM((_N_WORDS, _CCOLS), jnp.int32),
        pltpu.VMEM((_N_WORDS, _CCOLS), jnp.int32),
        pltpu.VMEM((_N_WORDS, _CCOLS), jnp.int32),
        pltpu.SemaphoreType.DMA,
        pltpu.SemaphoreType.DMA,
    ],
    compiler_params=pltpu.CompilerParams(needs_layout_passes=False),
)
def _lookup(tok_hbm, keys_hbm, out_hbm, table, keys_v, tok_a, tok_b, res_v,
            sem_a, sem_b):
    wid = lax.axis_index("s") * _NC + lax.axis_index("c")
    col0 = wid * _COLS_PER_W

    tok_bufs = (tok_a, tok_b)
    sems = (sem_a, sem_b)

    copies = [None] * _NCHUNK
    copies[0] = pltpu.make_async_copy(
        tok_hbm.at[:, pl.ds(col0, _CCOLS)], tok_a, sem_a
    )
    copies[0].start()

    pltpu.sync_copy(keys_hbm, keys_v)

    def init_body(j, carry):
        off, rv = carry
        for _ in range(_INIT_UNROLL):
            table[pl.ds(off, _L)] = rv
            rv2 = rv + jnp.int32(_L)
            rv = jnp.where(rv2 >= jnp.int32(_V + _N_OOV),
                           rv2 - jnp.int32(_N_OOV), rv2)
            off = off + jnp.int32(_L)
        return off, rv

    rv0 = jnp.int32(_V) + lax.iota(jnp.int32, _L)
    lax.fori_loop(0, _TOKEN_SPACE // _L // _INIT_UNROLL, init_body,
                  (jnp.int32(0), rv0))

    def scatter_body(j, off):
        lanes = lax.iota(jnp.int32, _L) + off
        ranks = jnp.minimum(lanes, jnp.int32(_V - 1))
        keys = keys_v[pl.ds(off, _L)]
        plsc.store_scatter(table, [keys], ranks)
        return off + jnp.int32(_L)

    lax.fori_loop(0, _VPAD // _L, scatter_body, jnp.int32(0))

    for c in range(_NCHUNK):
        tok_v = tok_bufs[c % 2]
        copies[c].wait()
        if c + 1 < _NCHUNK:
            copies[c + 1] = pltpu.make_async_copy(
                tok_hbm.at[:, pl.ds(col0 + (c + 1) * _CCOLS, _CCOLS)],
                tok_bufs[(c + 1) % 2],
                sems[(c + 1) % 2],
            )
            copies[c + 1].start()

        def w_body(w, woff):
            for kb in range(_CCOLS // _L):
                t = tok_v[woff, pl.ds(kb * _L, _L)]
                g = plsc.load_gather(table, [t])
                res_v[woff, pl.ds(kb * _L, _L)] = g
            return woff + jnp.int32(1)

        lax.fori_loop(0, _N_WORDS, w_body, jnp.int32(0))

        pltpu.sync_copy(res_v, out_hbm.at[:, pl.ds(col0 + c * _CCOLS, _CCOLS)])


def kernel(inputs, vocab_keys):
    tok = inputs.astype(jnp.int32).T
    keys = vocab_keys.astype(jnp.int32)
    keys = jnp.concatenate(
        [keys, jnp.full((_VPAD - _V,), keys[_V - 1], jnp.int32)]
    )
    out = _lookup(tok, keys).T
    return lax.bitcast_convert_type(out, jnp.uint32).astype(jnp.int64)


# speedup vs baseline: 2622.5285x; 1.0802x over previous
import functools

import jax
import jax.numpy as jnp
from jax import lax
from jax.experimental import pallas as pl
from jax.experimental.pallas import tpu as pltpu
from jax.experimental.pallas import tpu_sc as plsc

_BATCH = 16384
_N_WORDS = 50
_TOKEN_SPACE = 100000
_N_OOV = 100
_V = 1001
_VPAD = 1008
_L = 16
_NC = 2
_NS = 16
_NW = _NC * _NS
_COLS_PER_W = _BATCH // _NW
_CCOLS = 128
_NCHUNK = _COLS_PER_W // _CCOLS
_INIT_UNROLL = 10

_mesh = plsc.VectorSubcoreMesh(core_axis_name="c", subcore_axis_name="s")


@functools.partial(
    pl.kernel,
    mesh=_mesh,
    out_type=jax.ShapeDtypeStruct((_N_WORDS, _BATCH), jnp.int32),
    scratch_types=[
        pltpu.VMEM((_TOKEN_SPACE,), jnp.int32),
        pltpu.VMEM((_VPAD,), jnp.int32),
        pltpu.VMEM((_N_WORDS, _CCOLS), jnp.int32),
        pltpu.VMEM((_N_WORDS, _CCOLS), jnp.int32),
        pltpu.VMEM((_N_WORDS, _CCOLS), jnp.int32),
        pltpu.SemaphoreType.DMA,
        pltpu.SemaphoreType.DMA,
    ],
    compiler_params=pltpu.CompilerParams(needs_layout_passes=False),
)
def _lookup(tok_hbm, keys_hbm, out_hbm, table, keys_v, tok_a, tok_b, res_v,
            sem_a, sem_b):
    wid = lax.axis_index("s") * _NC + lax.axis_index("c")
    col0 = wid * _COLS_PER_W

    tok_bufs = (tok_a, tok_b)
    sems = (sem_a, sem_b)

    copies = [None] * _NCHUNK
    copies[0] = pltpu.make_async_copy(
        tok_hbm.at[:, pl.ds(col0, _CCOLS)], tok_a, sem_a
    )
    copies[0].start()

    pltpu.sync_copy(keys_hbm, keys_v)

    iota = lax.iota(jnp.int32, _L)

    @plsc.parallel_loop(jnp.int32(0), jnp.int32(_TOKEN_SPACE // _L),
                        jnp.int32(1), unroll=_INIT_UNROLL)
    def _(i):
        off = i * jnp.int32(_L)
        base = jnp.int32(_V) + lax.rem(off, jnp.int32(_N_OOV))
        rv = base + iota
        rv = jnp.where(rv >= jnp.int32(_V + _N_OOV),
                       rv - jnp.int32(_N_OOV), rv)
        table[pl.ds(off, _L)] = rv

    @plsc.parallel_loop(jnp.int32(0), jnp.int32(_VPAD // _L), jnp.int32(1),
                        unroll=4)
    def _(j):
        off = j * jnp.int32(_L)
        ranks = jnp.minimum(iota + off, jnp.int32(_V - 1))
        keys = keys_v[pl.ds(off, _L)]
        plsc.store_scatter(table, [keys], ranks)

    for c in range(_NCHUNK):
        tok_v = tok_bufs[c % 2]
        copies[c].wait()
        if c + 1 < _NCHUNK:
            copies[c + 1] = pltpu.make_async_copy(
                tok_hbm.at[:, pl.ds(col0 + (c + 1) * _CCOLS, _CCOLS)],
                tok_bufs[(c + 1) % 2],
                sems[(c + 1) % 2],
            )
            copies[c + 1].start()

        @plsc.parallel_loop(jnp.int32(0),
                            jnp.int32(_N_WORDS * (_CCOLS // _L)),
                            jnp.int32(1), unroll=8)
        def _(i):
            w = lax.div(i, jnp.int32(_CCOLS // _L))
            kb16 = lax.rem(i, jnp.int32(_CCOLS // _L)) * jnp.int32(_L)
            t = tok_v[w, pl.ds(kb16, _L)]
            g = plsc.load_gather(table, [t])
            res_v[w, pl.ds(kb16, _L)] = g

        pltpu.sync_copy(res_v, out_hbm.at[:, pl.ds(col0 + c * _CCOLS, _CCOLS)])


def kernel(inputs, vocab_keys):
    tok = inputs.astype(jnp.int32).T
    keys = vocab_keys.astype(jnp.int32)
    keys = jnp.concatenate(
        [keys, jnp.full((_VPAD - _V,), keys[_V - 1], jnp.int32)]
    )
    out = _lookup(tok, keys).T
    return lax.bitcast_convert_type(out, jnp.uint32).astype(jnp.int64)


# speedup vs baseline: 2662.5282x; 1.0153x over previous
import functools

import jax
import jax.numpy as jnp
from jax import lax
from jax.experimental import pallas as pl
from jax.experimental.pallas import tpu as pltpu
from jax.experimental.pallas import tpu_sc as plsc

_BATCH = 16384
_N_WORDS = 50
_TOKEN_SPACE = 100000
_N_OOV = 100
_V = 1001
_VPAD = 1008
_L = 16
_NC = 2
_NS = 16
_NW = _NC * _NS
_COLS_PER_W = _BATCH // _NW
_CCOLS = 128
_NCHUNK = _COLS_PER_W // _CCOLS
_INIT_UNROLL = 10

_mesh = plsc.VectorSubcoreMesh(core_axis_name="c", subcore_axis_name="s")


@functools.partial(
    pl.kernel,
    mesh=_mesh,
    out_type=jax.ShapeDtypeStruct((_N_WORDS, _BATCH), jnp.int32),
    scratch_types=[
        pltpu.VMEM((_TOKEN_SPACE,), jnp.int32),
        pltpu.VMEM((_VPAD,), jnp.int32),
        pltpu.VMEM((_N_WORDS, _CCOLS), jnp.int32),
        pltpu.VMEM((_N_WORDS, _CCOLS), jnp.int32),
        pltpu.VMEM((_N_WORDS, _CCOLS), jnp.int32),
        pltpu.SemaphoreType.DMA,
        pltpu.SemaphoreType.DMA,
    ],
    compiler_params=pltpu.CompilerParams(needs_layout_passes=False),
)
def _lookup(tok_hbm, keys_hbm, out_hbm, table, keys_v, tok_a, tok_b, res_v,
            sem_a, sem_b):
    wid = lax.axis_index("s") * _NC + lax.axis_index("c")
    col0 = wid * _COLS_PER_W

    tok_bufs = (tok_a, tok_b)
    sems = (sem_a, sem_b)

    copies = [None] * _NCHUNK
    copies[0] = pltpu.make_async_copy(
        tok_hbm.at[:, pl.ds(col0, _CCOLS)], tok_a, sem_a
    )
    copies[0].start()

    pltpu.sync_copy(keys_hbm, keys_v.at[pl.ds(0, _V)])

    iota = lax.iota(jnp.int32, _L)

    @plsc.parallel_loop(jnp.int32(0), jnp.int32(_TOKEN_SPACE // _L),
                        jnp.int32(1), unroll=_INIT_UNROLL)
    def _(i):
        off = i * jnp.int32(_L)
        base = jnp.int32(_V) + lax.rem(off, jnp.int32(_N_OOV))
        rv = base + iota
        rv = jnp.where(rv >= jnp.int32(_V + _N_OOV),
                       rv - jnp.int32(_N_OOV), rv)
        table[pl.ds(off, _L)] = rv

    @plsc.parallel_loop(jnp.int32(0), jnp.int32(_VPAD // _L), jnp.int32(1),
                        unroll=4)
    def _(j):
        off = j * jnp.int32(_L)
        lanes = iota + off
        keys = keys_v[pl.ds(off, _L)]
        plsc.store_scatter(table, [keys], lanes, mask=lanes < jnp.int32(_V))

    for c in range(_NCHUNK):
        tok_v = tok_bufs[c % 2]
        copies[c].wait()
        if c + 1 < _NCHUNK:
            copies[c + 1] = pltpu.make_async_copy(
                tok_hbm.at[:, pl.ds(col0 + (c + 1) * _CCOLS, _CCOLS)],
                tok_bufs[(c + 1) % 2],
                sems[(c + 1) % 2],
            )
            copies[c + 1].start()

        @plsc.parallel_loop(jnp.int32(0),
                            jnp.int32(_N_WORDS * (_CCOLS // _L)),
                            jnp.int32(1), unroll=8)
        def _(i):
            w = lax.div(i, jnp.int32(_CCOLS // _L))
            kb16 = lax.rem(i, jnp.int32(_CCOLS // _L)) * jnp.int32(_L)
            t = tok_v[w, pl.ds(kb16, _L)]
            g = plsc.load_gather(table, [t])
            res_v[w, pl.ds(kb16, _L)] = g

        pltpu.sync_copy(res_v, out_hbm.at[:, pl.ds(col0 + c * _CCOLS, _CCOLS)])


def kernel(inputs, vocab_keys):
    tok = inputs.astype(jnp.int32).T
    keys = vocab_keys.astype(jnp.int32)
    out = _lookup(tok, keys).T
    return lax.bitcast_convert_type(out, jnp.uint32).astype(jnp.int64)


# speedup vs baseline: 2822.6379x; 1.0601x over previous
import functools

import jax
import jax.numpy as jnp
from jax import lax
from jax.experimental import pallas as pl
from jax.experimental.pallas import tpu as pltpu
from jax.experimental.pallas import tpu_sc as plsc

_BATCH = 16384
_N_WORDS = 50
_TOKEN_SPACE = 100000
_N_OOV = 100
_V = 1001
_VPAD = 1008
_L = 16
_NC = 2
_NS = 16
_NW = _NC * _NS
_COLS_PER_W = _BATCH // _NW
_CCOLS = 128
_NCHUNK = _COLS_PER_W // _CCOLS
_INIT_UNROLL = 10

_mesh = plsc.VectorSubcoreMesh(core_axis_name="c", subcore_axis_name="s")


@functools.partial(
    pl.kernel,
    mesh=_mesh,
    out_type=jax.ShapeDtypeStruct((_N_WORDS, _BATCH), jnp.uint32),
    scratch_types=[
        pltpu.VMEM((_TOKEN_SPACE,), jnp.int32),
        pltpu.VMEM((_VPAD,), jnp.int32),
        pltpu.VMEM((_N_WORDS, _CCOLS), jnp.int32),
        pltpu.VMEM((_N_WORDS, _CCOLS), jnp.int32),
        pltpu.VMEM((_N_WORDS, _CCOLS), jnp.uint32),
        pltpu.SemaphoreType.DMA,
        pltpu.SemaphoreType.DMA,
    ],
    compiler_params=pltpu.CompilerParams(needs_layout_passes=False),
)
def _lookup(tok_hbm, keys_hbm, out_hbm, table, keys_v, tok_a, tok_b, res_v,
            sem_a, sem_b):
    wid = lax.axis_index("s") * _NC + lax.axis_index("c")
    col0 = wid * _COLS_PER_W

    tok_bufs = (tok_a, tok_b)
    sems = (sem_a, sem_b)

    copies = [None] * _NCHUNK
    copies[0] = pltpu.make_async_copy(
        tok_hbm.at[:, pl.ds(col0, _CCOLS)], tok_a, sem_a
    )
    copies[0].start()

    pltpu.sync_copy(keys_hbm, keys_v.at[pl.ds(0, _V)])

    iota = lax.iota(jnp.int32, _L)

    @plsc.parallel_loop(jnp.int32(0), jnp.int32(_TOKEN_SPACE // _L),
                        jnp.int32(1), unroll=_INIT_UNROLL)
    def _(i):
        off = i * jnp.int32(_L)
        base = jnp.int32(_V) + lax.rem(off, jnp.int32(_N_OOV))
        rv = base + iota
        rv = jnp.where(rv >= jnp.int32(_V + _N_OOV),
                       rv - jnp.int32(_N_OOV), rv)
        table[pl.ds(off, _L)] = rv

    @plsc.parallel_loop(jnp.int32(0), jnp.int32(_VPAD // _L), jnp.int32(1),
                        unroll=4)
    def _(j):
        off = j * jnp.int32(_L)
        lanes = iota + off
        keys = keys_v[pl.ds(off, _L)]
        plsc.store_scatter(table, [keys], lanes, mask=lanes < jnp.int32(_V))

    for c in range(_NCHUNK):
        tok_v = tok_bufs[c % 2]
        copies[c].wait()
        if c + 1 < _NCHUNK:
            copies[c + 1] = pltpu.make_async_copy(
                tok_hbm.at[:, pl.ds(col0 + (c + 1) * _CCOLS, _CCOLS)],
                tok_bufs[(c + 1) % 2],
                sems[(c + 1) % 2],
            )
            copies[c + 1].start()

        @plsc.parallel_loop(jnp.int32(0),
                            jnp.int32(_N_WORDS * (_CCOLS // _L)),
                            jnp.int32(1), unroll=8)
        def _(i):
            w = lax.div(i, jnp.int32(_CCOLS // _L))
            kb16 = lax.rem(i, jnp.int32(_CCOLS // _L)) * jnp.int32(_L)
            t = tok_v[w, pl.ds(kb16, _L)]
            g = plsc.load_gather(table, [t])
            res_v[w, pl.ds(kb16, _L)] = plsc.bitcast(g, jnp.uint32)

        pltpu.sync_copy(res_v, out_hbm.at[:, pl.ds(col0 + c * _CCOLS, _CCOLS)])


def kernel(inputs, vocab_keys):
    tok = inputs.astype(jnp.int32).T
    keys = vocab_keys.astype(jnp.int32)
    return _lookup(tok, keys).T.astype(jnp.int64)
